# baseline (speedup 1.0000x reference)
import functools

import jax
import jax.numpy as jnp
from jax.experimental import pallas as pl
from jax.experimental.pallas import tpu as pltpu


def _round_bf16(x):
    return x.astype(jnp.bfloat16).astype(jnp.float32)


def _mmh_block(are_ref, moves_ref, mask_ref, g_ref,
               wq1_ref, bq1_ref, wq2_ref, bq2_ref, wkey_ref, bkey_ref,
               logits_ref, policy_ref, idx_ref, *, n_moves):
    f32 = jnp.float32
    bt = are_ref.shape[0]
    e_dim = moves_ref.shape[-1]
    h = jnp.dot(are_ref[...], wq1_ref[...], preferred_element_type=f32)
    h = jnp.maximum(h + bq1_ref[...], 0.0)
    q = jnp.dot(h, wq2_ref[...], preferred_element_type=f32) + bq2_ref[...]
    moves2d = moves_ref[...].reshape(bt * n_moves, e_dim)
    keys = jnp.dot(moves2d, wkey_ref[...], preferred_element_type=f32)
    keys = keys + bkey_ref[...]
    qb = _round_bf16(q)
    kb = _round_bf16(keys).reshape(bt, n_moves, keys.shape[-1])
    cols = [jnp.sum(qb * kb[:, n, :], axis=-1, keepdims=True)
            for n in range(n_moves)]
    logits = jnp.concatenate(cols, axis=-1)
    logits_ref[...] = logits
    mask = mask_ref[...] > 0.0
    masked = jnp.where(mask, logits, -1e30)
    masked = masked - jnp.max(masked, axis=-1, keepdims=True)
    exps = jnp.where(mask, jnp.exp(masked), 0.0)
    policy = exps / jnp.sum(exps, axis=-1, keepdims=True)
    policy_ref[...] = policy
    score = jnp.log(policy + 1e-30) + g_ref[...]
    best = jnp.max(score, axis=-1, keepdims=True)
    ids = jax.lax.broadcasted_iota(jnp.int32, score.shape, 1)
    idx = jnp.min(jnp.where(score >= best, ids, n_moves), axis=-1,
                  keepdims=True)
    idx_ref[...] = idx


def kernel(action_type_index, autoregressive_embedding, max_moves,
           max_move_mask, W_key, b_key, W_q1, b_q1, W_q2, b_q2,
           W_proj, b_proj):
    T, B, S = autoregressive_embedding.shape
    N = max_moves.shape[-2]
    E = max_moves.shape[-1]
    K = W_key.shape[-1]
    TB = T * B

    are = autoregressive_embedding.reshape(TB, S)
    moves = max_moves.reshape(TB, N, E)
    mask = max_move_mask.reshape(TB, N)
    mask = jnp.where(jnp.sum(mask) == 0, jnp.ones_like(mask), mask)
    mask_f = mask.astype(jnp.float32)
    gumbel = jax.random.gumbel(jax.random.key(42), (TB, N), jnp.float32)
    bq1 = b_q1.reshape(1, K)
    bq2 = b_q2.reshape(1, K)
    bkey = b_key.reshape(1, K)

    BT = 256
    grid = (TB // BT,)

    tok = lambda i: (i, 0)
    rep = lambda i: (0, 0)

    logits, policy, idx = pl.pallas_call(
        functools.partial(_mmh_block, n_moves=N),
        grid=grid,
        in_specs=[
            pl.BlockSpec((BT, S), tok),
            pl.BlockSpec((BT, N, E), lambda i: (i, 0, 0)),
            pl.BlockSpec((BT, N), tok),
            pl.BlockSpec((BT, N), tok),
            pl.BlockSpec((S, K), rep),
            pl.BlockSpec((1, K), rep),
            pl.BlockSpec((K, K), rep),
            pl.BlockSpec((1, K), rep),
            pl.BlockSpec((E, K), rep),
            pl.BlockSpec((1, K), rep),
        ],
        out_specs=[
            pl.BlockSpec((BT, N), tok),
            pl.BlockSpec((BT, N), tok),
            pl.BlockSpec((BT, 1), tok),
        ],
        out_shape=[
            jax.ShapeDtypeStruct((TB, N), jnp.float32),
            jax.ShapeDtypeStruct((TB, N), jnp.float32),
            jax.ShapeDtypeStruct((TB, 1), jnp.int32),
        ],
        compiler_params=pltpu.CompilerParams(
            dimension_semantics=("arbitrary",),
        ),
    )(are, moves, mask_f, gumbel, W_q1, bq1, W_q2, bq2, W_key, bkey)

    return (logits.reshape(T, B, N), policy.reshape(T, B, N),
            idx.reshape(T, B, 1))
